# baseline (speedup 1.0000x reference)
import functools

import jax
import jax.numpy as jnp
from jax import lax
from jax.experimental import pallas as pl
from jax.experimental.pallas import tpu as pltpu
from jax.experimental.pallas import tpu_sc as plsc

D = 64
SCALE = 8.0
NC = 2
NS = 16
NW = NC * NS
CHUNK = 128
B_TOTAL = 4096 * 50
B_PER_W = B_TOTAL // NW
NCHUNK = B_PER_W // CHUNK

_mesh = plsc.VectorSubcoreMesh(core_axis_name="c", subcore_axis_name="s")


@functools.partial(
    pl.kernel,
    mesh=_mesh,
    out_type=jax.ShapeDtypeStruct((NW, NCHUNK, CHUNK, D), jnp.float32),
    scratch_types=[
        pltpu.VMEM((NCHUNK, CHUNK), jnp.int32),
        pltpu.VMEM((CHUNK, D), jnp.float32),
        pltpu.SemaphoreType.DMA,
    ],
    compiler_params=pltpu.CompilerParams(use_tc_tiling_on_sc=False),
)
def _emb_lookup(x_hbm, table_hbm, out_hbm, idx_v, rows_v, gsem):
    wid = lax.axis_index("s") * NC + lax.axis_index("c")
    pltpu.sync_copy(x_hbm.at[wid], idx_v)

    def chunk_body(j, carry):
        pltpu.async_copy(table_hbm.at[idx_v.at[j]], rows_v, gsem).wait()

        def scale_row(r, c2):
            for c in range(D // 16):
                sl = pl.ds(c * 16, 16)
                rows_v[r, sl] = rows_v[r, sl] * SCALE
            return c2

        lax.fori_loop(0, CHUNK, scale_row, 0)
        pltpu.sync_copy(rows_v, out_hbm.at[wid, j])
        return carry

    lax.fori_loop(0, NCHUNK, chunk_body, 0)


def kernel(x, table):
    xr = x.reshape(NW, NCHUNK, CHUNK)
    out = _emb_lookup(xr, table)
    return out.reshape(4096, 50, D)


# speedup vs baseline: 1.0813x; 1.0813x over previous
import functools

import jax
import jax.numpy as jnp
from jax import lax
from jax.experimental import pallas as pl
from jax.experimental.pallas import tpu as pltpu
from jax.experimental.pallas import tpu_sc as plsc

D = 64
SCALE = 8.0
NC = 2
NS = 16
NW = NC * NS
CHUNK = 128
B_TOTAL = 4096 * 50
B_PER_W = B_TOTAL // NW
NCHUNK = B_PER_W // CHUNK
NBUF = 5

_mesh = plsc.VectorSubcoreMesh(core_axis_name="c", subcore_axis_name="s")


@functools.partial(
    pl.kernel,
    mesh=_mesh,
    out_type=jax.ShapeDtypeStruct((NW, NCHUNK, CHUNK, D), jnp.float32),
    scratch_types=[
        pltpu.VMEM((NCHUNK, CHUNK), jnp.int32),
        pltpu.VMEM((NBUF, CHUNK, D), jnp.float32),
        pltpu.VMEM((NBUF, CHUNK, D), jnp.float32),
        pltpu.SemaphoreType.DMA((NBUF,)),
        pltpu.SemaphoreType.DMA((NBUF,)),
    ],
    compiler_params=pltpu.CompilerParams(use_tc_tiling_on_sc=False),
)
def _emb_lookup(x_hbm, table_hbm, out_hbm, idx_v, rows_v, outs_v, gsem, ssem):
    wid = lax.axis_index("s") * NC + lax.axis_index("c")
    pltpu.sync_copy(x_hbm.at[wid], idx_v)

    def gather_start(j, b):
        pltpu.async_copy(table_hbm.at[idx_v.at[j]], rows_v.at[b], gsem.at[b])

    def gather_wait(j, b):
        pltpu.make_async_copy(
            table_hbm.at[idx_v.at[j]], rows_v.at[b], gsem.at[b]
        ).wait()

    def store_start(j, b):
        pltpu.async_copy(outs_v.at[b], out_hbm.at[wid, j], ssem.at[b])

    def store_wait(j, b):
        pltpu.make_async_copy(
            outs_v.at[b], out_hbm.at[wid, j], ssem.at[b]
        ).wait()

    for b in range(NBUF):
        gather_start(b, b)

    def round_body(g, carry):
        for b in range(NBUF):
            j = g + b
            gather_wait(j, b)

            @pl.when(j >= NBUF)
            def _():
                store_wait(j - NBUF, b)

            @plsc.parallel_loop(0, CHUNK, unroll=4)
            def _(r):
                for c in range(D // 16):
                    sl = pl.ds(c * 16, 16)
                    outs_v[b, r, sl] = rows_v[b, r, sl] * SCALE

            @pl.when(j + NBUF < NCHUNK)
            def _():
                gather_start(j + NBUF, b)

            store_start(j, b)
        return carry

    lax.fori_loop(0, NCHUNK // NBUF, lambda i, c: round_body(i * NBUF, c), 0)

    for b in range(NBUF):
        store_wait(NCHUNK - NBUF + b, b)


def kernel(x, table):
    xr = x.reshape(NW, NCHUNK, CHUNK)
    out = _emb_lookup(xr, table)
    return out.reshape(4096, 50, D)


# speedup vs baseline: 1.0910x; 1.0090x over previous
import functools

import jax
import jax.numpy as jnp
from jax import lax
from jax.experimental import pallas as pl
from jax.experimental.pallas import tpu as pltpu
from jax.experimental.pallas import tpu_sc as plsc

D = 64
DPAD = 128
SCALE = 8.0
NC = 2
NS = 16
NW = NC * NS
CHUNK = 128
B_TOTAL = 4096 * 50
B_PER_W = B_TOTAL // NW
NCHUNK = B_PER_W // CHUNK
NBUF = 3

_mesh = plsc.VectorSubcoreMesh(core_axis_name="c", subcore_axis_name="s")


@functools.partial(
    pl.kernel,
    mesh=_mesh,
    out_type=jax.ShapeDtypeStruct((NW, NCHUNK, CHUNK, D), jnp.float32),
    scratch_types=[
        pltpu.VMEM((NCHUNK, CHUNK), jnp.int32),
        pltpu.VMEM((NBUF, CHUNK, DPAD), jnp.float32),
        pltpu.VMEM((NBUF, CHUNK, D), jnp.float32),
        pltpu.SemaphoreType.DMA((NBUF,)),
        pltpu.SemaphoreType.DMA((NBUF,)),
    ],
    compiler_params=pltpu.CompilerParams(use_tc_tiling_on_sc=True),
)
def _emb_lookup(x_hbm, table_hbm, out_hbm, idx_v, rows_v, outs_v, gsem, ssem):
    wid = lax.axis_index("s") * NC + lax.axis_index("c")
    pltpu.sync_copy(x_hbm.at[wid], idx_v)

    def gather_start(j, b):
        pltpu.async_copy(table_hbm.at[idx_v.at[j]], rows_v.at[b], gsem.at[b])

    def gather_wait(j, b):
        pltpu.make_async_copy(
            table_hbm.at[idx_v.at[j]], rows_v.at[b], gsem.at[b]
        ).wait()

    def store_start(j, b):
        pltpu.async_copy(outs_v.at[b], out_hbm.at[wid, j], ssem.at[b])

    def store_wait(j, b):
        pltpu.make_async_copy(
            outs_v.at[b], out_hbm.at[wid, j], ssem.at[b]
        ).wait()

    for b in range(NBUF):
        gather_start(b, b)

    def chunk_body(j, carry):
        b = lax.rem(j, NBUF)
        gather_wait(j, b)

        @pl.when(j >= NBUF)
        def _():
            store_wait(j - NBUF, b)

        @plsc.parallel_loop(0, CHUNK, unroll=4)
        def _(r):
            for c in range(D // 16):
                sl = pl.ds(c * 16, 16)
                outs_v[b, r, sl] = rows_v[b, r, sl] * SCALE

        @pl.when(j + NBUF < NCHUNK)
        def _():
            gather_start(j + NBUF, b)

        store_start(j, b)
        return carry

    lax.fori_loop(0, NCHUNK, chunk_body, 0)

    for k in range(NBUF):
        j = NCHUNK - NBUF + k
        store_wait(j, j % NBUF)


def kernel(x, table):
    xr = x.reshape(NW, NCHUNK, CHUNK)
    tpad = jnp.pad(table, ((0, 0), (0, DPAD - D)))
    out = _emb_lookup(xr, tpad)
    return out.reshape(4096, 50, D)


# speedup vs baseline: 1.1989x; 1.0988x over previous
import functools

import jax
import jax.numpy as jnp
from jax import lax
from jax.experimental import pallas as pl
from jax.experimental.pallas import tpu as pltpu
from jax.experimental.pallas import tpu_sc as plsc

D = 64
DPAD = 128
SCALE = 8.0
NC = 2
NS = 16
NW = NC * NS
SEQ = 50
NTOK = 4096
ROWS_PER_W = NTOK // NW
RB = 2
CHUNK = RB * SEQ
NCHUNK = ROWS_PER_W // RB
NBUF = 3

_mesh = plsc.VectorSubcoreMesh(core_axis_name="c", subcore_axis_name="s")


@functools.partial(
    pl.kernel,
    mesh=_mesh,
    out_type=jax.ShapeDtypeStruct((NTOK, SEQ, D), jnp.float32),
    scratch_types=[
        pltpu.VMEM((NCHUNK, CHUNK), jnp.int32),
        pltpu.VMEM((NBUF, CHUNK, DPAD), jnp.float32),
        pltpu.VMEM((NBUF, RB, SEQ, D), jnp.float32),
        pltpu.SemaphoreType.DMA((NBUF,)),
        pltpu.SemaphoreType.DMA((NBUF,)),
    ],
    compiler_params=pltpu.CompilerParams(use_tc_tiling_on_sc=True),
)
def _emb_lookup(x_hbm, table_hbm, out_hbm, idx_v, rows_v, outs_v, gsem, ssem):
    wid = lax.axis_index("s") * NC + lax.axis_index("c")
    base = wid * ROWS_PER_W
    pltpu.sync_copy(x_hbm.at[wid], idx_v)

    def gather_start(j, b):
        pltpu.async_copy(table_hbm.at[idx_v.at[j]], rows_v.at[b], gsem.at[b])

    def gather_wait(j, b):
        pltpu.make_async_copy(
            table_hbm.at[idx_v.at[j]], rows_v.at[b], gsem.at[b]
        ).wait()

    def store_start(j, b):
        pltpu.async_copy(
            outs_v.at[b], out_hbm.at[pl.ds(base + j * RB, RB)], ssem.at[b]
        )

    def store_wait(j, b):
        pltpu.make_async_copy(
            outs_v.at[b], out_hbm.at[pl.ds(base + j * RB, RB)], ssem.at[b]
        ).wait()

    for b in range(NBUF):
        gather_start(b, b)

    def chunk_body(j, carry):
        b = lax.rem(j, NBUF)
        gather_wait(j, b)

        @pl.when(j >= NBUF)
        def _():
            store_wait(j - NBUF, b)

        for db in range(RB):

            @plsc.parallel_loop(0, SEQ, unroll=5)
            def _(s):
                for c in range(D // 16):
                    sl = pl.ds(c * 16, 16)
                    outs_v[b, db, s, sl] = rows_v[b, db * SEQ + s, sl] * SCALE

        @pl.when(j + NBUF < NCHUNK)
        def _():
            gather_start(j + NBUF, b)

        store_start(j, b)
        return carry

    lax.fori_loop(0, NCHUNK, chunk_body, 0)

    for k in range(NBUF):
        j = NCHUNK - NBUF + k
        store_wait(j, j % NBUF)


def kernel(x, table):
    xr = x.reshape(NW, NCHUNK, CHUNK)
    tpad = jnp.pad(table, ((0, 0), (0, DPAD - D)))
    return _emb_lookup(xr, tpad)


# speedup vs baseline: 1.2033x; 1.0037x over previous
import functools

import jax
import jax.numpy as jnp
from jax import lax
from jax.experimental import pallas as pl
from jax.experimental.pallas import tpu as pltpu
from jax.experimental.pallas import tpu_sc as plsc

D = 64
DPAD = 128
SCALE = 8.0
NC = 2
NS = 16
NW = NC * NS
SEQ = 50
NTOK = 4096
ROWS_PER_W = NTOK // NW
RB = 2
CHUNK = RB * SEQ
NCHUNK = ROWS_PER_W // RB
NBUF = 4

_mesh = plsc.VectorSubcoreMesh(core_axis_name="c", subcore_axis_name="s")


@functools.partial(
    pl.kernel,
    mesh=_mesh,
    out_type=jax.ShapeDtypeStruct((NTOK, SEQ, D), jnp.float32),
    scratch_types=[
        pltpu.VMEM((NCHUNK, CHUNK), jnp.int32),
        pltpu.VMEM((NBUF, CHUNK, DPAD), jnp.float32),
        pltpu.VMEM((NBUF, RB, SEQ, D), jnp.float32),
        pltpu.SemaphoreType.DMA((NBUF,)),
        pltpu.SemaphoreType.DMA((NBUF,)),
    ],
    compiler_params=pltpu.CompilerParams(use_tc_tiling_on_sc=True),
)
def _emb_lookup(x_hbm, table_hbm, out_hbm, idx_v, rows_v, outs_v, gsem, ssem):
    wid = lax.axis_index("s") * NC + lax.axis_index("c")
    base = wid * ROWS_PER_W
    pltpu.sync_copy(x_hbm.at[wid], idx_v)

    def gather_start(j, b):
        pltpu.async_copy(table_hbm.at[idx_v.at[j]], rows_v.at[b], gsem.at[b])

    def gather_wait(j, b):
        pltpu.make_async_copy(
            table_hbm.at[idx_v.at[j]], rows_v.at[b], gsem.at[b]
        ).wait()

    def store_start(j, b):
        pltpu.async_copy(
            outs_v.at[b], out_hbm.at[pl.ds(base + j * RB, RB)], ssem.at[b]
        )

    def store_wait(j, b):
        pltpu.make_async_copy(
            outs_v.at[b], out_hbm.at[pl.ds(base + j * RB, RB)], ssem.at[b]
        ).wait()

    for b in range(NBUF):
        gather_start(b, b)

    def chunk_body(j, carry):
        b = lax.rem(j, NBUF)
        gather_wait(j, b)

        @pl.when(j >= NBUF)
        def _():
            store_wait(j - NBUF, b)

        for db in range(RB):

            @plsc.parallel_loop(0, SEQ, unroll=5)
            def _(s):
                for c in range(D // 16):
                    sl = pl.ds(c * 16, 16)
                    outs_v[b, db, s, sl] = rows_v[b, db * SEQ + s, sl] * SCALE

        @pl.when(j + NBUF < NCHUNK)
        def _():
            gather_start(j + NBUF, b)

        store_start(j, b)
        return carry

    lax.fori_loop(0, NCHUNK, chunk_body, 0)

    for k in range(NBUF):
        j = NCHUNK - NBUF + k
        store_wait(j, j % NBUF)


def kernel(x, table):
    xr = x.reshape(NW, NCHUNK, CHUNK)
    tpad = jnp.pad(table, ((0, 0), (0, DPAD - D)))
    return _emb_lookup(xr, tpad)


# speedup vs baseline: 1.2054x; 1.0017x over previous
import functools

import jax
import jax.numpy as jnp
from jax import lax
from jax.experimental import pallas as pl
from jax.experimental.pallas import tpu as pltpu
from jax.experimental.pallas import tpu_sc as plsc

D = 64
DPAD = 128
SCALE = 8.0
NC = 2
NS = 16
NW = NC * NS
SEQ = 50
NTOK = 4096
ROWS_PER_W = NTOK // NW
RB = 1
CHUNK = RB * SEQ
NCHUNK = ROWS_PER_W // RB
NBUF = 8

_mesh = plsc.VectorSubcoreMesh(core_axis_name="c", subcore_axis_name="s")


@functools.partial(
    pl.kernel,
    mesh=_mesh,
    out_type=jax.ShapeDtypeStruct((NTOK, SEQ, D), jnp.float32),
    scratch_types=[
        pltpu.VMEM((NCHUNK, CHUNK), jnp.int32),
        pltpu.VMEM((NBUF, CHUNK, DPAD), jnp.float32),
        pltpu.VMEM((NBUF, RB, SEQ, D), jnp.float32),
        pltpu.SemaphoreType.DMA((NBUF,)),
        pltpu.SemaphoreType.DMA((NBUF,)),
    ],
    compiler_params=pltpu.CompilerParams(use_tc_tiling_on_sc=True),
)
def _emb_lookup(x_hbm, table_hbm, out_hbm, idx_v, rows_v, outs_v, gsem, ssem):
    wid = lax.axis_index("s") * NC + lax.axis_index("c")
    base = wid * ROWS_PER_W
    pltpu.sync_copy(x_hbm.at[wid], idx_v)

    def gather_start(j, b):
        pltpu.async_copy(table_hbm.at[idx_v.at[j]], rows_v.at[b], gsem.at[b])

    def gather_wait(j, b):
        pltpu.make_async_copy(
            table_hbm.at[idx_v.at[j]], rows_v.at[b], gsem.at[b]
        ).wait()

    def store_start(j, b):
        pltpu.async_copy(
            outs_v.at[b], out_hbm.at[pl.ds(base + j * RB, RB)], ssem.at[b]
        )

    def store_wait(j, b):
        pltpu.make_async_copy(
            outs_v.at[b], out_hbm.at[pl.ds(base + j * RB, RB)], ssem.at[b]
        ).wait()

    for b in range(NBUF):
        gather_start(b, b)

    def chunk_body(j, carry):
        b = lax.rem(j, NBUF)
        gather_wait(j, b)

        @pl.when(j >= NBUF)
        def _():
            store_wait(j - NBUF, b)

        for db in range(RB):

            @plsc.parallel_loop(0, SEQ, unroll=5)
            def _(s):
                for c in range(D // 16):
                    sl = pl.ds(c * 16, 16)
                    outs_v[b, db, s, sl] = rows_v[b, db * SEQ + s, sl] * SCALE

        @pl.when(j + NBUF < NCHUNK)
        def _():
            gather_start(j + NBUF, b)

        store_start(j, b)
        return carry

    lax.fori_loop(0, NCHUNK, chunk_body, 0)

    for k in range(NBUF):
        j = NCHUNK - NBUF + k
        store_wait(j, j % NBUF)


def kernel(x, table):
    xr = x.reshape(NW, NCHUNK, CHUNK)
    tpad = jnp.pad(table, ((0, 0), (0, DPAD - D)))
    return _emb_lookup(xr, tpad)
